# Initial kernel scaffold; baseline (speedup 1.0000x reference)
import functools
import jax
import jax.numpy as jnp
import numpy as np
from jax import lax
from jax.experimental import pallas as pl
from jax.experimental.pallas import tpu as pltpu
from jax.experimental.pallas import tpu_sc as plsc

N = 10000
E = 320000
D = 128

NC = 2
NS = 16
NW = NC * NS
CHUNK = 128
NCHUNKS = E // CHUNK
JMAX = (NCHUNKS + NW - 1) // NW

_gelu = functools.partial(jax.nn.gelu, approximate=False)


def _dot(a, b):
    return jnp.dot(a, b, preferred_element_type=jnp.float32)


def _node_enc_body(x_ref, w_ref, b_ref, st_ref, ag_ref, vb_ref, p_ref):
    h = x_ref[...]
    for i in range(4):
        h = _dot(h, w_ref[i]) + b_ref[i, :][None, :]
        if i < 3:
            h = _gelu(h)
    vb = h * st_ref[0, :][None, :] + st_ref[1, :][None, :]
    vb_ref[...] = vb
    p_ref[...] = _dot(vb, ag_ref[...])


def _node_enc(x, w, b, st, ag, bn):
    grid = (N // bn,)
    full = lambda shape: pl.BlockSpec(shape, lambda i: (0,) * len(shape))
    return pl.pallas_call(
        _node_enc_body,
        grid=grid,
        in_specs=[
            pl.BlockSpec((bn, D), lambda i: (i, 0)),
            full((4, D, D)), full((4, D)), full((2, D)), full((D, D)),
        ],
        out_specs=[pl.BlockSpec((bn, D), lambda i: (i, 0))] * 2,
        out_shape=[jax.ShapeDtypeStruct((N, D), jnp.float32)] * 2,
        compiler_params=pltpu.CompilerParams(
            dimension_semantics=("arbitrary",)),
    )(x, w, b, st, ag)


def _edge_enc_body(ea_ref, w0_ref, w_ref, b_ref, ae_ref, ce_ref,
                   q0_ref, q1_ref, q2_ref):
    h = _gelu(_dot(ea_ref[...], w0_ref[...]) + b_ref[0, :][None, :])
    for i in range(3):
        h = _dot(h, w_ref[i]) + b_ref[i + 1, :][None, :]
        if i < 2:
            h = _gelu(h)
    for i, q_ref in enumerate((q0_ref, q1_ref, q2_ref)):
        q_ref[...] = _dot(h, ae_ref[i]) + ce_ref[i, :][None, :]


def _edge_enc(ea, w0, w, b, ae, ce, be):
    grid = (E // be,)
    full = lambda shape: pl.BlockSpec(shape, lambda i: (0,) * len(shape))
    return pl.pallas_call(
        _edge_enc_body,
        grid=grid,
        in_specs=[
            pl.BlockSpec((be, 16), lambda i: (i, 0)),
            full((16, D)), full((3, D, D)), full((4, D)),
            full((3, D, D)), full((3, D)),
        ],
        out_specs=[pl.BlockSpec((be, D), lambda i: (i, 0))] * 3,
        out_shape=[jax.ShapeDtypeStruct((E, D), jnp.float32)] * 3,
        compiler_params=pltpu.CompilerParams(
            dimension_semantics=("arbitrary",)),
    )(ea, w0, w, b, ae, ce)


def _mp_mlp_body(gs_ref, gd_ref, q_ref, w_ref, b_ref, m_ref):
    h = _gelu(gs_ref[...] + gd_ref[...] + q_ref[...])
    for i in range(3):
        h = _dot(h, w_ref[i]) + b_ref[i, :][None, :]
        if i < 2:
            h = _gelu(h)
    m_ref[...] = h


def _mp_mlp(gs, gd, q, w, b, be):
    grid = (E // be,)
    full = lambda shape: pl.BlockSpec(shape, lambda i: (0,) * len(shape))
    return pl.pallas_call(
        _mp_mlp_body,
        grid=grid,
        in_specs=[
            pl.BlockSpec((be, D), lambda i: (i, 0)),
            pl.BlockSpec((be, D), lambda i: (i, 0)),
            pl.BlockSpec((be, D), lambda i: (i, 0)),
            full((3, D, D)), full((3, D)),
        ],
        out_specs=pl.BlockSpec((be, D), lambda i: (i, 0)),
        out_shape=jax.ShapeDtypeStruct((E, D), jnp.float32),
        compiler_params=pltpu.CompilerParams(
            dimension_semantics=("arbitrary",)),
    )(gs, gd, q, w, b)


def _update_body(vb_ref, pa0_ref, pa1_ref, st_ref, ag_ref, vb2_ref, p_ref):
    u = vb_ref[...] + pa0_ref[...] + pa1_ref[...]
    vb = u * st_ref[0, :][None, :] + st_ref[1, :][None, :]
    vb2_ref[...] = vb
    p_ref[...] = _dot(vb, ag_ref[...])


def _update(vb, pa, st, ag, bn):
    grid = (N // bn,)
    nb = N // bn
    full = lambda shape: pl.BlockSpec(shape, lambda i: (0,) * len(shape))
    return pl.pallas_call(
        _update_body,
        grid=grid,
        in_specs=[
            pl.BlockSpec((bn, D), lambda i: (i, 0)),
            pl.BlockSpec((bn, D), lambda i: (i, 0)),
            pl.BlockSpec((bn, D), lambda i: (i + nb, 0)),
            full((2, D)), full((D, D)),
        ],
        out_specs=[pl.BlockSpec((bn, D), lambda i: (i, 0))] * 2,
        out_shape=[jax.ShapeDtypeStruct((N, D), jnp.float32)] * 2,
        compiler_params=pltpu.CompilerParams(
            dimension_semantics=("arbitrary",)),
    )(vb, pa, pa, st, ag)


def _final_body(vb_ref, pa0_ref, pa1_ref, w_ref, b_ref, f_ref):
    h = vb_ref[...] + pa0_ref[...] + pa1_ref[...]
    for i in range(4):
        h = _dot(h, w_ref[i]) + b_ref[i, :][None, :]
        if i < 3:
            h = _gelu(h)
    f_ref[...] = h


def _final(vb, pa, w, b, bn):
    grid = (N // bn,)
    nb = N // bn
    full = lambda shape: pl.BlockSpec(shape, lambda i: (0,) * len(shape))
    return pl.pallas_call(
        _final_body,
        grid=grid,
        in_specs=[
            pl.BlockSpec((bn, D), lambda i: (i, 0)),
            pl.BlockSpec((bn, D), lambda i: (i, 0)),
            pl.BlockSpec((bn, D), lambda i: (i + nb, 0)),
            full((4, D, D)), full((4, D)),
        ],
        out_specs=pl.BlockSpec((bn, D), lambda i: (i, 0)),
        out_shape=jax.ShapeDtypeStruct((N, D), jnp.float32),
        compiler_params=pltpu.CompilerParams(
            dimension_semantics=("arbitrary",)),
    )(vb, pa, pa, w, b)


_SC_MESH = plsc.VectorSubcoreMesh(
    core_axis_name="c", subcore_axis_name="s", num_cores=NC, num_subcores=NS)


def _sc_gather_body(p_hbm, src_hbm, dst_hbm, gs_hbm, gd_hbm,
                    idx_s, idx_d, rows_s, rows_d, sem_s, sem_d):
    wid = lax.axis_index("s") * NC + lax.axis_index("c")

    def step(j, carry):
        cid = wid + NW * j

        @pl.when(cid < NCHUNKS)
        def _():
            base = cid * CHUNK
            pltpu.sync_copy(src_hbm.at[pl.ds(base, CHUNK)], idx_s)
            pltpu.sync_copy(dst_hbm.at[pl.ds(base, CHUNK)], idx_d)
            cp_s = pltpu.async_copy(p_hbm.at[idx_s], rows_s, sem_s)
            cp_d = pltpu.async_copy(p_hbm.at[idx_d], rows_d, sem_d)
            cp_s.wait()
            cp_d.wait()
            pltpu.sync_copy(rows_s, gs_hbm.at[pl.ds(base, CHUNK)])
            pltpu.sync_copy(rows_d, gd_hbm.at[pl.ds(base, CHUNK)])
        return carry

    lax.fori_loop(0, JMAX, step, 0)


_sc_gather = pl.kernel(
    _sc_gather_body,
    out_type=[jax.ShapeDtypeStruct((E, D), jnp.float32)] * 2,
    mesh=_SC_MESH,
    scratch_types=[
        pltpu.VMEM((CHUNK,), jnp.int32),
        pltpu.VMEM((CHUNK,), jnp.int32),
        pltpu.VMEM((CHUNK, D), jnp.float32),
        pltpu.VMEM((CHUNK, D), jnp.float32),
        pltpu.SemaphoreType.DMA,
        pltpu.SemaphoreType.DMA,
    ],
)


def _sc_scatter_body(m_hbm, dst_hbm, zeros_hbm, pa_hbm,
                     acc, idx_v, m_v):
    cid = lax.axis_index("c")
    sid = lax.axis_index("s")
    wid = sid * NC + cid
    rows_per_tile = N // NS

    r0 = sid * rows_per_tile
    pltpu.sync_copy(zeros_hbm.at[pl.ds(r0, rows_per_tile)],
                    acc.at[pl.ds(r0, rows_per_tile)])
    plsc.subcore_barrier()

    def step(j, carry):
        chunk = wid + NW * j

        @pl.when(chunk < NCHUNKS)
        def _():
            base = chunk * CHUNK
            pltpu.sync_copy(dst_hbm.at[pl.ds(base, CHUNK)], idx_v)
            pltpu.sync_copy(m_hbm.at[pl.ds(base, CHUNK)], m_v)
            pltpu.sync_copy(m_v, acc.at[idx_v], add=True)
        return carry

    lax.fori_loop(0, JMAX, step, 0)
    plsc.subcore_barrier()

    pltpu.sync_copy(acc.at[pl.ds(r0, rows_per_tile)],
                    pa_hbm.at[pl.ds(cid * N + r0, rows_per_tile)])


_sc_scatter = pl.kernel(
    _sc_scatter_body,
    out_type=jax.ShapeDtypeStruct((2 * N, D), jnp.float32),
    mesh=_SC_MESH,
    scratch_types=[
        pltpu.VMEM_SHARED((N, D), jnp.float32),
        pltpu.VMEM((CHUNK,), jnp.int32),
        pltpu.VMEM((CHUNK, D), jnp.float32),
    ],
)


def kernel(x, edge_index, edge_attr, params):
    f32 = jnp.float32
    s = params['bn_w'] * f32(1.0 / np.sqrt(1.0 + 1e-5))
    t = params['bn_b']
    st = jnp.stack([s, t])
    src = edge_index[0].astype(jnp.int32)
    dst = edge_index[1].astype(jnp.int32)

    node_w = jnp.stack([W.T for W, _ in params['node_enc']])
    node_b = jnp.stack([b for _, b in params['node_enc']])
    enc_w0 = params['edge_enc'][0][0].T
    enc_w = jnp.stack([W.T for W, _ in params['edge_enc'][1:]])
    enc_b = jnp.stack([b for _, b in params['edge_enc']])
    dec_w = jnp.stack([W.T for W, _ in params['dec']])
    dec_b = jnp.stack([b for _, b in params['dec']])

    ags, aes, ces, mp_w, mp_b = [], [], [], [], []
    for i in range(3):
        W0, b0 = params['mp%d' % i][0]
        ags.append(W0[:, :D].T)
        aes.append(s[:, None] * W0[:, D:].T)
        ces.append(t @ W0[:, D:].T + b0)
        mp_w.append(jnp.stack([W.T for W, _ in params['mp%d' % i][1:]]))
        mp_b.append(jnp.stack([b for _, b in params['mp%d' % i][1:]]))
    aes = jnp.stack(aes)
    ces = jnp.stack(ces)

    zeros = jnp.zeros((N, D), f32)

    vb, p = _node_enc(x, node_w, node_b, st, ags[0], bn=1000)
    qs = _edge_enc(edge_attr, enc_w0, enc_w, enc_b, aes, ces, be=1600)

    for i in range(3):
        gs, gd = _sc_gather(p, src, dst)
        m = _mp_mlp(gs, gd, qs[i], mp_w[i], mp_b[i], be=1600)
        pa = _sc_scatter(m, dst, zeros)
        if i < 2:
            vb, p = _update(vb, pa, st, ags[i + 1], bn=1000)
        else:
            f = _final(vb, pa, dec_w, dec_b, bn=1000)
    return f


# speedup vs baseline: 4.0161x; 4.0161x over previous
import functools
import jax
import jax.numpy as jnp
import numpy as np
from jax import lax
from jax.experimental import pallas as pl
from jax.experimental.pallas import tpu as pltpu
from jax.experimental.pallas import tpu_sc as plsc

N = 10000
E = 320000
D = 128

NC = 2
NS = 16
NW = NC * NS
CHUNK = 128
NCHUNKS = E // CHUNK
JMAX = (NCHUNKS + NW - 1) // NW

def _gelu(x):
    return 0.5 * x * (1.0 + lax.erf(x * np.float32(1.0 / np.sqrt(2.0))))


def _dot(a, b):
    return jnp.dot(a, b, preferred_element_type=jnp.float32)


def _node_enc_body(x_ref, w_ref, b_ref, st_ref, ag_ref, vb_ref, p_ref):
    h = x_ref[...]
    for i in range(4):
        h = _dot(h, w_ref[i]) + b_ref[i, :][None, :]
        if i < 3:
            h = _gelu(h)
    vb = h * st_ref[0, :][None, :] + st_ref[1, :][None, :]
    vb_ref[...] = vb
    p_ref[...] = _dot(vb, ag_ref[...])


def _node_enc(x, w, b, st, ag, bn):
    grid = (N // bn,)
    full = lambda shape: pl.BlockSpec(shape, lambda i: (0,) * len(shape))
    return pl.pallas_call(
        _node_enc_body,
        grid=grid,
        in_specs=[
            pl.BlockSpec((bn, D), lambda i: (i, 0)),
            full((4, D, D)), full((4, D)), full((2, D)), full((D, D)),
        ],
        out_specs=[pl.BlockSpec((bn, D), lambda i: (i, 0))] * 2,
        out_shape=[jax.ShapeDtypeStruct((N, D), jnp.float32)] * 2,
        compiler_params=pltpu.CompilerParams(
            dimension_semantics=("arbitrary",)),
    )(x, w, b, st, ag)


def _edge_enc_body(ea_ref, w0_ref, w_ref, b_ref, ae_ref, ce_ref,
                   q0_ref, q1_ref, q2_ref):
    h = _gelu(_dot(ea_ref[...], w0_ref[...]) + b_ref[0, :][None, :])
    for i in range(3):
        h = _dot(h, w_ref[i]) + b_ref[i + 1, :][None, :]
        if i < 2:
            h = _gelu(h)
    for i, q_ref in enumerate((q0_ref, q1_ref, q2_ref)):
        q_ref[...] = _dot(h, ae_ref[i]) + ce_ref[i, :][None, :]


def _edge_enc(ea, w0, w, b, ae, ce, be):
    grid = (E // be,)
    full = lambda shape: pl.BlockSpec(shape, lambda i: (0,) * len(shape))
    return pl.pallas_call(
        _edge_enc_body,
        grid=grid,
        in_specs=[
            pl.BlockSpec((be, 16), lambda i: (i, 0)),
            full((16, D)), full((3, D, D)), full((4, D)),
            full((3, D, D)), full((3, D)),
        ],
        out_specs=[pl.BlockSpec((be, D), lambda i: (i, 0))] * 3,
        out_shape=[jax.ShapeDtypeStruct((E, D), jnp.float32)] * 3,
        compiler_params=pltpu.CompilerParams(
            dimension_semantics=("arbitrary",)),
    )(ea, w0, w, b, ae, ce)


def _mp_mlp_body(gs_ref, gd_ref, q_ref, w_ref, b_ref, m_ref):
    h = _gelu(gs_ref[...] + gd_ref[...] + q_ref[...])
    for i in range(3):
        h = _dot(h, w_ref[i]) + b_ref[i, :][None, :]
        if i < 2:
            h = _gelu(h)
    m_ref[...] = h


def _mp_mlp(gs, gd, q, w, b, be):
    grid = (E // be,)
    full = lambda shape: pl.BlockSpec(shape, lambda i: (0,) * len(shape))
    return pl.pallas_call(
        _mp_mlp_body,
        grid=grid,
        in_specs=[
            pl.BlockSpec((be, D), lambda i: (i, 0)),
            pl.BlockSpec((be, D), lambda i: (i, 0)),
            pl.BlockSpec((be, D), lambda i: (i, 0)),
            full((3, D, D)), full((3, D)),
        ],
        out_specs=pl.BlockSpec((be, D), lambda i: (i, 0)),
        out_shape=jax.ShapeDtypeStruct((E, D), jnp.float32),
        compiler_params=pltpu.CompilerParams(
            dimension_semantics=("arbitrary",)),
    )(gs, gd, q, w, b)


def _update_body(vb_ref, pa0_ref, pa1_ref, st_ref, ag_ref, vb2_ref, p_ref):
    u = vb_ref[...] + pa0_ref[...] + pa1_ref[...]
    vb = u * st_ref[0, :][None, :] + st_ref[1, :][None, :]
    vb2_ref[...] = vb
    p_ref[...] = _dot(vb, ag_ref[...])


def _update(vb, pa, st, ag, bn):
    grid = (N // bn,)
    nb = N // bn
    full = lambda shape: pl.BlockSpec(shape, lambda i: (0,) * len(shape))
    return pl.pallas_call(
        _update_body,
        grid=grid,
        in_specs=[
            pl.BlockSpec((bn, D), lambda i: (i, 0)),
            pl.BlockSpec((bn, D), lambda i: (i, 0)),
            pl.BlockSpec((bn, D), lambda i: (i + nb, 0)),
            full((2, D)), full((D, D)),
        ],
        out_specs=[pl.BlockSpec((bn, D), lambda i: (i, 0))] * 2,
        out_shape=[jax.ShapeDtypeStruct((N, D), jnp.float32)] * 2,
        compiler_params=pltpu.CompilerParams(
            dimension_semantics=("arbitrary",)),
    )(vb, pa, pa, st, ag)


def _final_body(vb_ref, pa0_ref, pa1_ref, w_ref, b_ref, f_ref):
    h = vb_ref[...] + pa0_ref[...] + pa1_ref[...]
    for i in range(4):
        h = _dot(h, w_ref[i]) + b_ref[i, :][None, :]
        if i < 3:
            h = _gelu(h)
    f_ref[...] = h


def _final(vb, pa, w, b, bn):
    grid = (N // bn,)
    nb = N // bn
    full = lambda shape: pl.BlockSpec(shape, lambda i: (0,) * len(shape))
    return pl.pallas_call(
        _final_body,
        grid=grid,
        in_specs=[
            pl.BlockSpec((bn, D), lambda i: (i, 0)),
            pl.BlockSpec((bn, D), lambda i: (i, 0)),
            pl.BlockSpec((bn, D), lambda i: (i + nb, 0)),
            full((4, D, D)), full((4, D)),
        ],
        out_specs=pl.BlockSpec((bn, D), lambda i: (i, 0)),
        out_shape=jax.ShapeDtypeStruct((N, D), jnp.float32),
        compiler_params=pltpu.CompilerParams(
            dimension_semantics=("arbitrary",)),
    )(vb, pa, pa, w, b)


@functools.cache
def _sc_mesh():
    return plsc.VectorSubcoreMesh(
        core_axis_name="c", subcore_axis_name="s",
        num_cores=NC, num_subcores=NS)


def _sc_gather_body(p_hbm, src_hbm, dst_hbm, gs_hbm, gd_hbm,
                    idx_s, idx_d, rows_s, rows_d, sem_s, sem_d):
    wid = lax.axis_index("s") * NC + lax.axis_index("c")

    def step(j, carry):
        cid = wid + NW * j

        @pl.when(cid < NCHUNKS)
        def _():
            base = cid * CHUNK
            pltpu.sync_copy(src_hbm.at[pl.ds(base, CHUNK)], idx_s)
            pltpu.sync_copy(dst_hbm.at[pl.ds(base, CHUNK)], idx_d)
            cp_s = pltpu.async_copy(p_hbm.at[idx_s], rows_s, sem_s)
            cp_d = pltpu.async_copy(p_hbm.at[idx_d], rows_d, sem_d)
            cp_s.wait()
            cp_d.wait()
            pltpu.sync_copy(rows_s, gs_hbm.at[pl.ds(base, CHUNK)])
            pltpu.sync_copy(rows_d, gd_hbm.at[pl.ds(base, CHUNK)])
        return carry

    lax.fori_loop(0, JMAX, step, 0)


@functools.cache
def _sc_gather():
    return pl.kernel(
        _sc_gather_body,
        out_type=[jax.ShapeDtypeStruct((E, D), jnp.float32)] * 2,
        mesh=_sc_mesh(),
        scratch_types=[
            pltpu.VMEM((CHUNK,), jnp.int32),
            pltpu.VMEM((CHUNK,), jnp.int32),
            pltpu.VMEM((CHUNK, D), jnp.float32),
            pltpu.VMEM((CHUNK, D), jnp.float32),
            pltpu.SemaphoreType.DMA,
            pltpu.SemaphoreType.DMA,
        ],
    )


def _sc_scatter_body(m_hbm, dst_hbm, zeros_hbm, pa_hbm,
                     acc, idx_v, m_v):
    cid = lax.axis_index("c")
    sid = lax.axis_index("s")
    wid = sid * NC + cid
    r0 = sid * 632

    @pl.when(sid < NS - 1)
    def _():
        pltpu.sync_copy(zeros_hbm.at[pl.ds(r0, 632)], acc.at[pl.ds(r0, 632)])

    @pl.when(sid == NS - 1)
    def _():
        pltpu.sync_copy(zeros_hbm.at[pl.ds(15 * 632, 520)],
                        acc.at[pl.ds(15 * 632, 520)])

    plsc.subcore_barrier()

    def step(j, carry):
        chunk = wid + NW * j

        @pl.when(chunk < NCHUNKS)
        def _():
            base = chunk * CHUNK
            pltpu.sync_copy(dst_hbm.at[pl.ds(base, CHUNK)], idx_v)
            pltpu.sync_copy(m_hbm.at[pl.ds(base, CHUNK)], m_v)
            pltpu.sync_copy(m_v, acc.at[idx_v], add=True)
        return carry

    lax.fori_loop(0, JMAX, step, 0)
    plsc.subcore_barrier()

    @pl.when(sid < NS - 1)
    def _():
        pltpu.sync_copy(acc.at[pl.ds(r0, 632)],
                        pa_hbm.at[pl.ds(cid * N + r0, 632)])

    @pl.when(sid == NS - 1)
    def _():
        pltpu.sync_copy(acc.at[pl.ds(15 * 632, 520)],
                        pa_hbm.at[pl.ds(cid * N + 15 * 632, 520)])


@functools.cache
def _sc_scatter():
    return pl.kernel(
        _sc_scatter_body,
        out_type=jax.ShapeDtypeStruct((2 * N, D), jnp.float32),
        mesh=_sc_mesh(),
        scratch_types=[
            pltpu.VMEM_SHARED((N, D), jnp.float32),
            pltpu.VMEM((CHUNK,), jnp.int32),
            pltpu.VMEM((CHUNK, D), jnp.float32),
        ],
    )


def kernel(x, edge_index, edge_attr, params):
    f32 = jnp.float32
    s = params['bn_w'] * f32(1.0 / np.sqrt(1.0 + 1e-5))
    t = params['bn_b']
    st = jnp.stack([s, t])
    src = edge_index[0].astype(jnp.int32)
    dst = edge_index[1].astype(jnp.int32)

    node_w = jnp.stack([W.T for W, _ in params['node_enc']])
    node_b = jnp.stack([b for _, b in params['node_enc']])
    enc_w0 = params['edge_enc'][0][0].T
    enc_w = jnp.stack([W.T for W, _ in params['edge_enc'][1:]])
    enc_b = jnp.stack([b for _, b in params['edge_enc']])
    dec_w = jnp.stack([W.T for W, _ in params['dec']])
    dec_b = jnp.stack([b for _, b in params['dec']])

    ags, aes, ces, mp_w, mp_b = [], [], [], [], []
    for i in range(3):
        W0, b0 = params['mp%d' % i][0]
        ags.append(W0[:, :D].T)
        aes.append(s[:, None] * W0[:, D:].T)
        ces.append(t @ W0[:, D:].T + b0)
        mp_w.append(jnp.stack([W.T for W, _ in params['mp%d' % i][1:]]))
        mp_b.append(jnp.stack([b for _, b in params['mp%d' % i][1:]]))
    aes = jnp.stack(aes)
    ces = jnp.stack(ces)

    zeros = jnp.zeros((N, D), f32)

    vb, p = _node_enc(x, node_w, node_b, st, ags[0], bn=1000)
    qs = _edge_enc(edge_attr, enc_w0, enc_w, enc_b, aes, ces, be=1600)

    for i in range(3):
        gs, gd = _sc_gather()(p, src, dst)
        m = _mp_mlp(gs, gd, qs[i], mp_w[i], mp_b[i], be=1600)
        pa = _sc_scatter()(m, dst, zeros)
        if i < 2:
            vb, p = _update(vb, pa, st, ags[i + 1], bn=1000)
        else:
            f = _final(vb, pa, dec_w, dec_b, bn=1000)
    return f


# speedup vs baseline: 4.5684x; 1.1375x over previous
import functools
import jax
import jax.numpy as jnp
import numpy as np
from jax import lax
from jax.experimental import pallas as pl
from jax.experimental.pallas import tpu as pltpu
from jax.experimental.pallas import tpu_sc as plsc

N = 10000
E = 320000
D = 128

NC = 2
NS = 16
NW = NC * NS
CHUNK = 128
NCHUNKS = E // CHUNK
JMAX = (NCHUNKS + NW - 1) // NW

def _gelu(x):
    return 0.5 * x * (1.0 + lax.erf(x * np.float32(1.0 / np.sqrt(2.0))))


def _dot(a, b):
    return jnp.dot(a, b, preferred_element_type=jnp.float32)


def _node_enc_body(x_ref, w_ref, b_ref, st_ref, ag_ref, vb_ref, p_ref):
    h = x_ref[...]
    for i in range(4):
        h = _dot(h, w_ref[i]) + b_ref[i, :][None, :]
        if i < 3:
            h = _gelu(h)
    vb = h * st_ref[0, :][None, :] + st_ref[1, :][None, :]
    vb_ref[...] = vb
    p_ref[...] = _dot(vb, ag_ref[...])


def _node_enc(x, w, b, st, ag, bn):
    grid = (N // bn,)
    full = lambda shape: pl.BlockSpec(shape, lambda i: (0,) * len(shape))
    return pl.pallas_call(
        _node_enc_body,
        grid=grid,
        in_specs=[
            pl.BlockSpec((bn, D), lambda i: (i, 0)),
            full((4, D, D)), full((4, D)), full((2, D)), full((D, D)),
        ],
        out_specs=[pl.BlockSpec((bn, D), lambda i: (i, 0))] * 2,
        out_shape=[jax.ShapeDtypeStruct((N, D), jnp.float32)] * 2,
        compiler_params=pltpu.CompilerParams(
            dimension_semantics=("arbitrary",)),
    )(x, w, b, st, ag)


def _edge_enc_body(ea_ref, w0_ref, w_ref, b_ref, ae_ref, ce_ref,
                   q0_ref, q1_ref, q2_ref):
    h = _gelu(_dot(ea_ref[...], w0_ref[...]) + b_ref[0, :][None, :])
    for i in range(3):
        h = _dot(h, w_ref[i]) + b_ref[i + 1, :][None, :]
        if i < 2:
            h = _gelu(h)
    for i, q_ref in enumerate((q0_ref, q1_ref, q2_ref)):
        q_ref[...] = _dot(h, ae_ref[i]) + ce_ref[i, :][None, :]


def _edge_enc(ea, w0, w, b, ae, ce, be):
    grid = (E // be,)
    full = lambda shape: pl.BlockSpec(shape, lambda i: (0,) * len(shape))
    return pl.pallas_call(
        _edge_enc_body,
        grid=grid,
        in_specs=[
            pl.BlockSpec((be, 16), lambda i: (i, 0)),
            full((16, D)), full((3, D, D)), full((4, D)),
            full((3, D, D)), full((3, D)),
        ],
        out_specs=[pl.BlockSpec((be, D), lambda i: (i, 0))] * 3,
        out_shape=[jax.ShapeDtypeStruct((E, D), jnp.float32)] * 3,
        compiler_params=pltpu.CompilerParams(
            dimension_semantics=("arbitrary",)),
    )(ea, w0, w, b, ae, ce)


def _mp_mlp_body(gs_ref, gd_ref, q_ref, w_ref, b_ref, m_ref):
    h = _gelu(gs_ref[...] + gd_ref[...] + q_ref[...])
    for i in range(3):
        h = _dot(h, w_ref[i]) + b_ref[i, :][None, :]
        if i < 2:
            h = _gelu(h)
    m_ref[...] = h


def _mp_mlp(gs, gd, q, w, b, be):
    grid = (E // be,)
    full = lambda shape: pl.BlockSpec(shape, lambda i: (0,) * len(shape))
    return pl.pallas_call(
        _mp_mlp_body,
        grid=grid,
        in_specs=[
            pl.BlockSpec((be, D), lambda i: (i, 0)),
            pl.BlockSpec((be, D), lambda i: (i, 0)),
            pl.BlockSpec((be, D), lambda i: (i, 0)),
            full((3, D, D)), full((3, D)),
        ],
        out_specs=pl.BlockSpec((be, D), lambda i: (i, 0)),
        out_shape=jax.ShapeDtypeStruct((E, D), jnp.float32),
        compiler_params=pltpu.CompilerParams(
            dimension_semantics=("arbitrary",)),
    )(gs, gd, q, w, b)


def _update_body(vb_ref, pa0_ref, pa1_ref, st_ref, ag_ref, vb2_ref, p_ref):
    u = vb_ref[...] + pa0_ref[...] + pa1_ref[...]
    vb = u * st_ref[0, :][None, :] + st_ref[1, :][None, :]
    vb2_ref[...] = vb
    p_ref[...] = _dot(vb, ag_ref[...])


def _update(vb, pa, st, ag, bn):
    grid = (N // bn,)
    nb = N // bn
    full = lambda shape: pl.BlockSpec(shape, lambda i: (0,) * len(shape))
    return pl.pallas_call(
        _update_body,
        grid=grid,
        in_specs=[
            pl.BlockSpec((bn, D), lambda i: (i, 0)),
            pl.BlockSpec((bn, D), lambda i: (i, 0)),
            pl.BlockSpec((bn, D), lambda i: (i + nb, 0)),
            full((2, D)), full((D, D)),
        ],
        out_specs=[pl.BlockSpec((bn, D), lambda i: (i, 0))] * 2,
        out_shape=[jax.ShapeDtypeStruct((N, D), jnp.float32)] * 2,
        compiler_params=pltpu.CompilerParams(
            dimension_semantics=("arbitrary",)),
    )(vb, pa, pa, st, ag)


def _final_body(vb_ref, pa0_ref, pa1_ref, w_ref, b_ref, f_ref):
    h = vb_ref[...] + pa0_ref[...] + pa1_ref[...]
    for i in range(4):
        h = _dot(h, w_ref[i]) + b_ref[i, :][None, :]
        if i < 3:
            h = _gelu(h)
    f_ref[...] = h


def _final(vb, pa, w, b, bn):
    grid = (N // bn,)
    nb = N // bn
    full = lambda shape: pl.BlockSpec(shape, lambda i: (0,) * len(shape))
    return pl.pallas_call(
        _final_body,
        grid=grid,
        in_specs=[
            pl.BlockSpec((bn, D), lambda i: (i, 0)),
            pl.BlockSpec((bn, D), lambda i: (i, 0)),
            pl.BlockSpec((bn, D), lambda i: (i + nb, 0)),
            full((4, D, D)), full((4, D)),
        ],
        out_specs=pl.BlockSpec((bn, D), lambda i: (i, 0)),
        out_shape=jax.ShapeDtypeStruct((N, D), jnp.float32),
        compiler_params=pltpu.CompilerParams(
            dimension_semantics=("arbitrary",)),
    )(vb, pa, pa, w, b)


@functools.cache
def _sc_mesh():
    return plsc.VectorSubcoreMesh(
        core_axis_name="c", subcore_axis_name="s",
        num_cores=NC, num_subcores=NS)


_TILE_ROWS_BIG = 632
_TILE_ROWS_LAST = N - (NS - 1) * _TILE_ROWS_BIG


def _per_tile_node_copy(sid, src_at, dst_at):
    r0 = sid * _TILE_ROWS_BIG

    @pl.when(sid < NS - 1)
    def _():
        pltpu.sync_copy(src_at(r0, _TILE_ROWS_BIG), dst_at(r0, _TILE_ROWS_BIG))

    @pl.when(sid == NS - 1)
    def _():
        r1 = (NS - 1) * _TILE_ROWS_BIG
        pltpu.sync_copy(src_at(r1, _TILE_ROWS_LAST), dst_at(r1, _TILE_ROWS_LAST))


KMAX = (JMAX + 1) // 2


def _sc_gather_body(p_hbm, src_hbm, dst_hbm, gs_hbm, gd_hbm,
                    idx_s0, idx_d0, idx_s1, idx_d1,
                    rs0, rd0, rs1, rd1,
                    gsem0, gsem1, wsem0, wsem1):
    cid_ax = lax.axis_index("c")
    sid = lax.axis_index("s")
    wid = sid * NC + cid_ax

    slots = ((idx_s0, idx_d0, rs0, rd0, gsem0, wsem0),
             (idx_s1, idx_d1, rs1, rd1, gsem1, wsem1))

    def start(c, slot):
        idxs, idxd, rs, rd, gsem, _ = slot
        base = c * CHUNK
        pltpu.sync_copy(src_hbm.at[pl.ds(base, CHUNK)], idxs)
        pltpu.sync_copy(dst_hbm.at[pl.ds(base, CHUNK)], idxd)
        pltpu.async_copy(p_hbm.at[idxs], rs, gsem)
        pltpu.async_copy(p_hbm.at[idxd], rd, gsem)

    def finish(c, slot):
        idxs, idxd, rs, rd, gsem, wsem = slot
        base = c * CHUNK
        pltpu.make_async_copy(p_hbm.at[idxs], rs, gsem).wait()
        pltpu.make_async_copy(p_hbm.at[idxd], rd, gsem).wait()
        pltpu.async_copy(rs, gs_hbm.at[pl.ds(base, CHUNK)], wsem)
        pltpu.async_copy(rd, gd_hbm.at[pl.ds(base, CHUNK)], wsem)

    def drain_writes(c, slot):
        _, _, rs, rd, _, wsem = slot
        base = c * CHUNK
        pltpu.make_async_copy(rs, gs_hbm.at[pl.ds(base, CHUNK)], wsem).wait()
        pltpu.make_async_copy(rd, gd_hbm.at[pl.ds(base, CHUNK)], wsem).wait()

    def step(k, carry):
        c0 = wid + NW * (2 * k)
        c1 = wid + NW * (2 * k + 1)
        cp0 = wid + NW * (2 * k - 2)
        cp1 = wid + NW * (2 * k - 1)

        @pl.when(jnp.logical_and(k > 0, cp0 < NCHUNKS))
        def _():
            drain_writes(cp0, slots[0])

        @pl.when(c0 < NCHUNKS)
        def _():
            start(c0, slots[0])

        @pl.when(jnp.logical_and(k > 0, cp1 < NCHUNKS))
        def _():
            drain_writes(cp1, slots[1])

        @pl.when(c1 < NCHUNKS)
        def _():
            start(c1, slots[1])

        @pl.when(c0 < NCHUNKS)
        def _():
            finish(c0, slots[0])

        @pl.when(c1 < NCHUNKS)
        def _():
            finish(c1, slots[1])
        return carry

    lax.fori_loop(0, KMAX, step, 0)

    ct0 = wid + NW * (2 * KMAX - 2)
    ct1 = wid + NW * (2 * KMAX - 1)

    @pl.when(ct0 < NCHUNKS)
    def _():
        drain_writes(ct0, slots[0])

    @pl.when(ct1 < NCHUNKS)
    def _():
        drain_writes(ct1, slots[1])


@functools.cache
def _sc_gather():
    return pl.kernel(
        _sc_gather_body,
        out_type=[jax.ShapeDtypeStruct((E, D), jnp.float32)] * 2,
        mesh=_sc_mesh(),
        scratch_types=[
            pltpu.VMEM((CHUNK,), jnp.int32),
            pltpu.VMEM((CHUNK,), jnp.int32),
            pltpu.VMEM((CHUNK,), jnp.int32),
            pltpu.VMEM((CHUNK,), jnp.int32),
            pltpu.VMEM((CHUNK, D), jnp.float32),
            pltpu.VMEM((CHUNK, D), jnp.float32),
            pltpu.VMEM((CHUNK, D), jnp.float32),
            pltpu.VMEM((CHUNK, D), jnp.float32),
            pltpu.SemaphoreType.DMA,
            pltpu.SemaphoreType.DMA,
            pltpu.SemaphoreType.DMA,
            pltpu.SemaphoreType.DMA,
        ],
    )


def _sc_scatter_body(m_hbm, dst_hbm, zeros_hbm, pa_hbm,
                     acc, idx0, idx1, m0, m1, asem0, asem1):
    cid = lax.axis_index("c")
    sid = lax.axis_index("s")
    wid = sid * NC + cid

    _per_tile_node_copy(sid,
                        lambda o, n: zeros_hbm.at[pl.ds(o, n)],
                        lambda o, n: acc.at[pl.ds(o, n)])
    plsc.subcore_barrier()

    slots = ((idx0, m0, asem0), (idx1, m1, asem1))

    def start(c, slot):
        idx, m_v, asem = slot
        base = c * CHUNK
        pltpu.sync_copy(dst_hbm.at[pl.ds(base, CHUNK)], idx)
        pltpu.sync_copy(m_hbm.at[pl.ds(base, CHUNK)], m_v)
        pltpu.async_copy(m_v, acc.at[idx], asem, add=True)

    def drain(slot):
        idx, m_v, asem = slot
        pltpu.make_async_copy(m_v, acc.at[idx], asem).wait()

    def step(k, carry):
        c0 = wid + NW * (2 * k)
        c1 = wid + NW * (2 * k + 1)

        @pl.when(jnp.logical_and(k > 0, wid + NW * (2 * k - 2) < NCHUNKS))
        def _():
            drain(slots[0])

        @pl.when(c0 < NCHUNKS)
        def _():
            start(c0, slots[0])

        @pl.when(jnp.logical_and(k > 0, wid + NW * (2 * k - 1) < NCHUNKS))
        def _():
            drain(slots[1])

        @pl.when(c1 < NCHUNKS)
        def _():
            start(c1, slots[1])
        return carry

    lax.fori_loop(0, KMAX, step, 0)

    @pl.when(wid + NW * (2 * KMAX - 2) < NCHUNKS)
    def _():
        drain(slots[0])

    @pl.when(wid + NW * (2 * KMAX - 1) < NCHUNKS)
    def _():
        drain(slots[1])

    plsc.subcore_barrier()

    _per_tile_node_copy(sid,
                        lambda o, n: acc.at[pl.ds(o, n)],
                        lambda o, n: pa_hbm.at[pl.ds(cid * N + o, n)])


@functools.cache
def _sc_scatter():
    return pl.kernel(
        _sc_scatter_body,
        out_type=jax.ShapeDtypeStruct((2 * N, D), jnp.float32),
        mesh=_sc_mesh(),
        scratch_types=[
            pltpu.VMEM_SHARED((N, D), jnp.float32),
            pltpu.VMEM((CHUNK,), jnp.int32),
            pltpu.VMEM((CHUNK,), jnp.int32),
            pltpu.VMEM((CHUNK, D), jnp.float32),
            pltpu.VMEM((CHUNK, D), jnp.float32),
            pltpu.SemaphoreType.DMA,
            pltpu.SemaphoreType.DMA,
        ],
    )


def kernel(x, edge_index, edge_attr, params):
    f32 = jnp.float32
    s = params['bn_w'] * f32(1.0 / np.sqrt(1.0 + 1e-5))
    t = params['bn_b']
    st = jnp.stack([s, t])
    src = edge_index[0].astype(jnp.int32)
    dst = edge_index[1].astype(jnp.int32)

    node_w = jnp.stack([W.T for W, _ in params['node_enc']])
    node_b = jnp.stack([b for _, b in params['node_enc']])
    enc_w0 = params['edge_enc'][0][0].T
    enc_w = jnp.stack([W.T for W, _ in params['edge_enc'][1:]])
    enc_b = jnp.stack([b for _, b in params['edge_enc']])
    dec_w = jnp.stack([W.T for W, _ in params['dec']])
    dec_b = jnp.stack([b for _, b in params['dec']])

    ags, aes, ces, mp_w, mp_b = [], [], [], [], []
    for i in range(3):
        W0, b0 = params['mp%d' % i][0]
        ags.append(W0[:, :D].T)
        aes.append(s[:, None] * W0[:, D:].T)
        ces.append(t @ W0[:, D:].T + b0)
        mp_w.append(jnp.stack([W.T for W, _ in params['mp%d' % i][1:]]))
        mp_b.append(jnp.stack([b for _, b in params['mp%d' % i][1:]]))
    aes = jnp.stack(aes)
    ces = jnp.stack(ces)

    zeros = jnp.zeros((N, D), f32)

    vb, p = _node_enc(x, node_w, node_b, st, ags[0], bn=1000)
    qs = _edge_enc(edge_attr, enc_w0, enc_w, enc_b, aes, ces, be=1600)

    for i in range(3):
        gs, gd = _sc_gather()(p, src, dst)
        m = _mp_mlp(gs, gd, qs[i], mp_w[i], mp_b[i], be=1600)
        pa = _sc_scatter()(m, dst, zeros)
        if i < 2:
            vb, p = _update(vb, pa, st, ags[i + 1], bn=1000)
        else:
            f = _final(vb, pa, dec_w, dec_b, bn=1000)
    return f


# speedup vs baseline: 5.0400x; 1.1032x over previous
import functools
import jax
import jax.numpy as jnp
import numpy as np
from jax import lax
from jax.experimental import pallas as pl
from jax.experimental.pallas import tpu as pltpu
from jax.experimental.pallas import tpu_sc as plsc

N = 10000
E = 320000
D = 128

NC = 2
NS = 16
NW = NC * NS
CHUNK = 128
NCHUNKS = E // CHUNK
JMAX = (NCHUNKS + NW - 1) // NW

def _gelu(x):
    return 0.5 * x * (1.0 + lax.erf(x * np.float32(1.0 / np.sqrt(2.0))))


def _dot(a, b):
    return jnp.dot(a, b, preferred_element_type=jnp.float32)


def _node_enc_body(x_ref, w_ref, b_ref, st_ref, ag_ref, vb_ref, p_ref):
    h = x_ref[...]
    for i in range(4):
        h = _dot(h, w_ref[i]) + b_ref[i, :][None, :]
        if i < 3:
            h = _gelu(h)
    vb = h * st_ref[0, :][None, :] + st_ref[1, :][None, :]
    vb_ref[...] = vb
    p_ref[...] = _dot(vb, ag_ref[...])


def _node_enc(x, w, b, st, ag, bn):
    grid = (N // bn,)
    full = lambda shape: pl.BlockSpec(shape, lambda i: (0,) * len(shape))
    return pl.pallas_call(
        _node_enc_body,
        grid=grid,
        in_specs=[
            pl.BlockSpec((bn, D), lambda i: (i, 0)),
            full((4, D, D)), full((4, D)), full((2, D)), full((D, D)),
        ],
        out_specs=[pl.BlockSpec((bn, D), lambda i: (i, 0))] * 2,
        out_shape=[jax.ShapeDtypeStruct((N, D), jnp.float32)] * 2,
        compiler_params=pltpu.CompilerParams(
            dimension_semantics=("arbitrary",)),
    )(x, w, b, st, ag)


def _edge_enc_body(ea_ref, w0_ref, w_ref, b_ref, ae_ref, ce_ref,
                   q0_ref, q1_ref, q2_ref):
    h = _gelu(_dot(ea_ref[...], w0_ref[...]) + b_ref[0, :][None, :])
    for i in range(3):
        h = _dot(h, w_ref[i]) + b_ref[i + 1, :][None, :]
        if i < 2:
            h = _gelu(h)
    for i, q_ref in enumerate((q0_ref, q1_ref, q2_ref)):
        q_ref[...] = _dot(h, ae_ref[i]) + ce_ref[i, :][None, :]


def _edge_enc(ea, w0, w, b, ae, ce, be):
    grid = (E // be,)
    full = lambda shape: pl.BlockSpec(shape, lambda i: (0,) * len(shape))
    return pl.pallas_call(
        _edge_enc_body,
        grid=grid,
        in_specs=[
            pl.BlockSpec((be, 16), lambda i: (i, 0)),
            full((16, D)), full((3, D, D)), full((4, D)),
            full((3, D, D)), full((3, D)),
        ],
        out_specs=[pl.BlockSpec((be, D), lambda i: (i, 0))] * 3,
        out_shape=[jax.ShapeDtypeStruct((E, D), jnp.float32)] * 3,
        compiler_params=pltpu.CompilerParams(
            dimension_semantics=("arbitrary",)),
    )(ea, w0, w, b, ae, ce)


def _mp_mlp_body(gs_ref, gd_ref, q_ref, w_ref, b_ref, m_ref):
    h = _gelu(gs_ref[...] + gd_ref[...] + q_ref[...])
    for i in range(3):
        h = _dot(h, w_ref[i]) + b_ref[i, :][None, :]
        if i < 2:
            h = _gelu(h)
    m_ref[...] = h


def _mp_mlp(gs, gd, q, qoff, w, b, be):
    esz = gs.shape[0]
    grid = (esz // be,)
    qb = qoff // be
    full = lambda shape: pl.BlockSpec(shape, lambda i: (0,) * len(shape))
    return pl.pallas_call(
        _mp_mlp_body,
        grid=grid,
        in_specs=[
            pl.BlockSpec((be, D), lambda i: (i, 0)),
            pl.BlockSpec((be, D), lambda i: (i, 0)),
            pl.BlockSpec((be, D), lambda i: (i + qb, 0)),
            full((3, D, D)), full((3, D)),
        ],
        out_specs=pl.BlockSpec((be, D), lambda i: (i, 0)),
        out_shape=jax.ShapeDtypeStruct((esz, D), jnp.float32),
        compiler_params=pltpu.CompilerParams(
            dimension_semantics=("arbitrary",)),
    )(gs, gd, q, w, b)


def _update_body(vb_ref, pa0_ref, pa1_ref, pb0_ref, pb1_ref,
                 st_ref, ag_ref, vb2_ref, p_ref):
    u = (vb_ref[...] + pa0_ref[...] + pa1_ref[...]
         + pb0_ref[...] + pb1_ref[...])
    vb = u * st_ref[0, :][None, :] + st_ref[1, :][None, :]
    vb2_ref[...] = vb
    p_ref[...] = _dot(vb, ag_ref[...])


def _update(vb, pa, pb, st, ag, bn):
    grid = (N // bn,)
    nb = N // bn
    full = lambda shape: pl.BlockSpec(shape, lambda i: (0,) * len(shape))
    return pl.pallas_call(
        _update_body,
        grid=grid,
        in_specs=[
            pl.BlockSpec((bn, D), lambda i: (i, 0)),
            pl.BlockSpec((bn, D), lambda i: (i, 0)),
            pl.BlockSpec((bn, D), lambda i: (i + nb, 0)),
            pl.BlockSpec((bn, D), lambda i: (i, 0)),
            pl.BlockSpec((bn, D), lambda i: (i + nb, 0)),
            full((2, D)), full((D, D)),
        ],
        out_specs=[pl.BlockSpec((bn, D), lambda i: (i, 0))] * 2,
        out_shape=[jax.ShapeDtypeStruct((N, D), jnp.float32)] * 2,
        compiler_params=pltpu.CompilerParams(
            dimension_semantics=("arbitrary",)),
    )(vb, pa, pa, pb, pb, st, ag)


def _final_body(vb_ref, pa0_ref, pa1_ref, pb0_ref, pb1_ref,
                w_ref, b_ref, f_ref):
    h = (vb_ref[...] + pa0_ref[...] + pa1_ref[...]
         + pb0_ref[...] + pb1_ref[...])
    for i in range(4):
        h = _dot(h, w_ref[i]) + b_ref[i, :][None, :]
        if i < 3:
            h = _gelu(h)
    f_ref[...] = h


def _final(vb, pa, pb, w, b, bn):
    grid = (N // bn,)
    nb = N // bn
    full = lambda shape: pl.BlockSpec(shape, lambda i: (0,) * len(shape))
    return pl.pallas_call(
        _final_body,
        grid=grid,
        in_specs=[
            pl.BlockSpec((bn, D), lambda i: (i, 0)),
            pl.BlockSpec((bn, D), lambda i: (i, 0)),
            pl.BlockSpec((bn, D), lambda i: (i + nb, 0)),
            pl.BlockSpec((bn, D), lambda i: (i, 0)),
            pl.BlockSpec((bn, D), lambda i: (i + nb, 0)),
            full((4, D, D)), full((4, D)),
        ],
        out_specs=pl.BlockSpec((bn, D), lambda i: (i, 0)),
        out_shape=jax.ShapeDtypeStruct((N, D), jnp.float32),
        compiler_params=pltpu.CompilerParams(
            dimension_semantics=("arbitrary",)),
    )(vb, pa, pa, pb, pb, w, b)


@functools.cache
def _sc_mesh():
    return plsc.VectorSubcoreMesh(
        core_axis_name="c", subcore_axis_name="s",
        num_cores=NC, num_subcores=NS)


_TILE_ROWS_BIG = 632
_TILE_ROWS_LAST = N - (NS - 1) * _TILE_ROWS_BIG


def _per_tile_node_copy(sid, src_at, dst_at):
    r0 = sid * _TILE_ROWS_BIG

    @pl.when(sid < NS - 1)
    def _():
        pltpu.sync_copy(src_at(r0, _TILE_ROWS_BIG), dst_at(r0, _TILE_ROWS_BIG))

    @pl.when(sid == NS - 1)
    def _():
        r1 = (NS - 1) * _TILE_ROWS_BIG
        pltpu.sync_copy(src_at(r1, _TILE_ROWS_LAST), dst_at(r1, _TILE_ROWS_LAST))


KMAX = (JMAX + 1) // 2


def _make_sc_gather_body(eoff, nchunks, kmax):
  def _sc_gather_body(p_hbm, src_hbm, dst_hbm, gs_hbm, gd_hbm,
                      idx_s0, idx_d0, idx_s1, idx_d1,
                      rs0, rd0, rs1, rd1,
                      gsem0, gsem1, wsem0, wsem1):
    NCHUNKS = nchunks
    KMAX = kmax
    cid_ax = lax.axis_index("c")
    sid = lax.axis_index("s")
    wid = sid * NC + cid_ax

    slots = ((idx_s0, idx_d0, rs0, rd0, gsem0, wsem0),
             (idx_s1, idx_d1, rs1, rd1, gsem1, wsem1))

    def start(c, slot):
        idxs, idxd, rs, rd, gsem, _ = slot
        base = c * CHUNK
        pltpu.sync_copy(src_hbm.at[pl.ds(eoff + base, CHUNK)], idxs)
        pltpu.sync_copy(dst_hbm.at[pl.ds(eoff + base, CHUNK)], idxd)
        pltpu.async_copy(p_hbm.at[idxs], rs, gsem)
        pltpu.async_copy(p_hbm.at[idxd], rd, gsem)

    def finish(c, slot):
        idxs, idxd, rs, rd, gsem, wsem = slot
        base = c * CHUNK
        pltpu.make_async_copy(p_hbm.at[idxs], rs, gsem).wait()
        pltpu.make_async_copy(p_hbm.at[idxd], rd, gsem).wait()
        pltpu.async_copy(rs, gs_hbm.at[pl.ds(base, CHUNK)], wsem)
        pltpu.async_copy(rd, gd_hbm.at[pl.ds(base, CHUNK)], wsem)

    def drain_writes(c, slot):
        _, _, rs, rd, _, wsem = slot
        base = c * CHUNK
        pltpu.make_async_copy(rs, gs_hbm.at[pl.ds(base, CHUNK)], wsem).wait()
        pltpu.make_async_copy(rd, gd_hbm.at[pl.ds(base, CHUNK)], wsem).wait()

    def step(k, carry):
        c0 = wid + NW * (2 * k)
        c1 = wid + NW * (2 * k + 1)
        cp0 = wid + NW * (2 * k - 2)
        cp1 = wid + NW * (2 * k - 1)

        @pl.when(jnp.logical_and(k > 0, cp0 < NCHUNKS))
        def _():
            drain_writes(cp0, slots[0])

        @pl.when(c0 < NCHUNKS)
        def _():
            start(c0, slots[0])

        @pl.when(jnp.logical_and(k > 0, cp1 < NCHUNKS))
        def _():
            drain_writes(cp1, slots[1])

        @pl.when(c1 < NCHUNKS)
        def _():
            start(c1, slots[1])

        @pl.when(c0 < NCHUNKS)
        def _():
            finish(c0, slots[0])

        @pl.when(c1 < NCHUNKS)
        def _():
            finish(c1, slots[1])
        return carry

    lax.fori_loop(0, KMAX, step, 0)

    ct0 = wid + NW * (2 * KMAX - 2)
    ct1 = wid + NW * (2 * KMAX - 1)

    @pl.when(ct0 < NCHUNKS)
    def _():
        drain_writes(ct0, slots[0])

    @pl.when(ct1 < NCHUNKS)
    def _():
        drain_writes(ct1, slots[1])

  return _sc_gather_body


@functools.cache
def _sc_gather(eoff, esz):
    nchunks = esz // CHUNK
    jmax = (nchunks + NW - 1) // NW
    kmax = (jmax + 1) // 2
    return pl.kernel(
        _make_sc_gather_body(eoff, nchunks, kmax),
        out_type=[jax.ShapeDtypeStruct((esz, D), jnp.float32)] * 2,
        mesh=_sc_mesh(),
        scratch_types=[
            pltpu.VMEM((CHUNK,), jnp.int32),
            pltpu.VMEM((CHUNK,), jnp.int32),
            pltpu.VMEM((CHUNK,), jnp.int32),
            pltpu.VMEM((CHUNK,), jnp.int32),
            pltpu.VMEM((CHUNK, D), jnp.float32),
            pltpu.VMEM((CHUNK, D), jnp.float32),
            pltpu.VMEM((CHUNK, D), jnp.float32),
            pltpu.VMEM((CHUNK, D), jnp.float32),
            pltpu.SemaphoreType.DMA,
            pltpu.SemaphoreType.DMA,
            pltpu.SemaphoreType.DMA,
            pltpu.SemaphoreType.DMA,
        ],
    )


def _make_sc_scatter_body(eoff, nchunks, kmax):
  def _sc_scatter_body(m_hbm, dst_hbm, zeros_hbm, pa_hbm,
                       acc, idx0, idx1, m0, m1, asem0, asem1):
    NCHUNKS = nchunks
    KMAX = kmax
    cid = lax.axis_index("c")
    sid = lax.axis_index("s")
    wid = sid * NC + cid

    _per_tile_node_copy(sid,
                        lambda o, n: zeros_hbm.at[pl.ds(o, n)],
                        lambda o, n: acc.at[pl.ds(o, n)])
    plsc.subcore_barrier()

    slots = ((idx0, m0, asem0), (idx1, m1, asem1))

    def start(c, slot):
        idx, m_v, asem = slot
        base = c * CHUNK
        pltpu.sync_copy(dst_hbm.at[pl.ds(eoff + base, CHUNK)], idx)
        pltpu.sync_copy(m_hbm.at[pl.ds(base, CHUNK)], m_v)
        pltpu.async_copy(m_v, acc.at[idx], asem, add=True)

    def drain(slot):
        idx, m_v, asem = slot
        pltpu.make_async_copy(m_v, acc.at[idx], asem).wait()

    def step(k, carry):
        c0 = wid + NW * (2 * k)
        c1 = wid + NW * (2 * k + 1)

        @pl.when(jnp.logical_and(k > 0, wid + NW * (2 * k - 2) < NCHUNKS))
        def _():
            drain(slots[0])

        @pl.when(c0 < NCHUNKS)
        def _():
            start(c0, slots[0])

        @pl.when(jnp.logical_and(k > 0, wid + NW * (2 * k - 1) < NCHUNKS))
        def _():
            drain(slots[1])

        @pl.when(c1 < NCHUNKS)
        def _():
            start(c1, slots[1])
        return carry

    lax.fori_loop(0, KMAX, step, 0)

    @pl.when(wid + NW * (2 * KMAX - 2) < NCHUNKS)
    def _():
        drain(slots[0])

    @pl.when(wid + NW * (2 * KMAX - 1) < NCHUNKS)
    def _():
        drain(slots[1])

    plsc.subcore_barrier()

    _per_tile_node_copy(sid,
                        lambda o, n: acc.at[pl.ds(o, n)],
                        lambda o, n: pa_hbm.at[pl.ds(cid * N + o, n)])

  return _sc_scatter_body


@functools.cache
def _sc_scatter(eoff, esz):
    nchunks = esz // CHUNK
    jmax = (nchunks + NW - 1) // NW
    kmax = (jmax + 1) // 2
    return pl.kernel(
        _make_sc_scatter_body(eoff, nchunks, kmax),
        out_type=jax.ShapeDtypeStruct((2 * N, D), jnp.float32),
        mesh=_sc_mesh(),
        scratch_types=[
            pltpu.VMEM_SHARED((N, D), jnp.float32),
            pltpu.VMEM((CHUNK,), jnp.int32),
            pltpu.VMEM((CHUNK,), jnp.int32),
            pltpu.VMEM((CHUNK, D), jnp.float32),
            pltpu.VMEM((CHUNK, D), jnp.float32),
            pltpu.SemaphoreType.DMA,
            pltpu.SemaphoreType.DMA,
        ],
    )


def kernel(x, edge_index, edge_attr, params):
    f32 = jnp.float32
    s = params['bn_w'] * f32(1.0 / np.sqrt(1.0 + 1e-5))
    t = params['bn_b']
    st = jnp.stack([s, t])
    src = edge_index[0].astype(jnp.int32)
    dst = edge_index[1].astype(jnp.int32)

    node_w = jnp.stack([W.T for W, _ in params['node_enc']])
    node_b = jnp.stack([b for _, b in params['node_enc']])
    enc_w0 = params['edge_enc'][0][0].T
    enc_w = jnp.stack([W.T for W, _ in params['edge_enc'][1:]])
    enc_b = jnp.stack([b for _, b in params['edge_enc']])
    dec_w = jnp.stack([W.T for W, _ in params['dec']])
    dec_b = jnp.stack([b for _, b in params['dec']])

    ags, aes, ces, mp_w, mp_b = [], [], [], [], []
    for i in range(3):
        W0, b0 = params['mp%d' % i][0]
        ags.append(W0[:, :D].T)
        aes.append(s[:, None] * W0[:, D:].T)
        ces.append(t @ W0[:, D:].T + b0)
        mp_w.append(jnp.stack([W.T for W, _ in params['mp%d' % i][1:]]))
        mp_b.append(jnp.stack([b for _, b in params['mp%d' % i][1:]]))
    aes = jnp.stack(aes)
    ces = jnp.stack(ces)

    zeros = jnp.zeros((N, D), f32)

    vb, p = _node_enc(x, node_w, node_b, st, ags[0], bn=1000)
    qs = _edge_enc(edge_attr, enc_w0, enc_w, enc_b, aes, ces, be=1600)

    EH = E // 2
    for i in range(3):
        gs0, gd0 = _sc_gather(0, EH)(p, src, dst)
        gs1, gd1 = _sc_gather(EH, EH)(p, src, dst)
        m0 = _mp_mlp(gs0, gd0, qs[i], 0, mp_w[i], mp_b[i], be=1600)
        pa0 = _sc_scatter(0, EH)(m0, dst, zeros)
        m1 = _mp_mlp(gs1, gd1, qs[i], EH, mp_w[i], mp_b[i], be=1600)
        pa1 = _sc_scatter(EH, EH)(m1, dst, zeros)
        if i < 2:
            vb, p = _update(vb, pa0, pa1, st, ags[i + 1], bn=1000)
        else:
            f = _final(vb, pa0, pa1, dec_w, dec_b, bn=1000)
    return f
